# Initial kernel scaffold; baseline (speedup 1.0000x reference)
import jax
import jax.numpy as jnp
from jax.experimental import pallas as pl


def kernel(x, edge_index, W, b):
    raise NotImplementedError("write your pallas kernel here")


# speedup vs baseline: 39.4575x; 39.4575x over previous
import functools

import jax
import jax.numpy as jnp
from jax import lax
from jax.experimental import pallas as pl
from jax.experimental.pallas import tpu as pltpu
from jax.experimental.pallas import tpu_sc as plsc

N = 10000
E = 320000
D = 128
NC = 2
NS = 16
NW = NC * NS
EPW = E // NW
WIN = 80
NWIN = EPW // WIN
ROWS_PER_TILE = 1000

_MESH = plsc.VectorSubcoreMesh(core_axis_name="c", subcore_axis_name="s")


def _deg_body(col_hbm, zeros_hbm, cnt_hbm, colv, ones_v, deg_sh, sem):
    cid = lax.axis_index("c")
    sid = lax.axis_index("s")
    wid = sid * NC + cid

    @pl.when(sid == 0)
    def _():
        pltpu.sync_copy(zeros_hbm, deg_sh)

    for i in range(8):
        ones_v[pl.ds(i * 16, 16)] = jnp.ones((16,), jnp.float32)
    pltpu.sync_copy(col_hbm.at[wid], colv)
    plsc.subcore_barrier()

    def body(w, carry):
        pltpu.sync_copy(ones_v.at[pl.ds(0, WIN)], deg_sh.at[colv.at[w]],
                        add=True)
        return carry

    lax.fori_loop(0, NWIN, body, 0)
    plsc.subcore_barrier()

    @pl.when(sid == 0)
    def _():
        pltpu.sync_copy(deg_sh, cnt_hbm.at[cid])


_deg = pl.kernel(
    _deg_body,
    out_type=jax.ShapeDtypeStruct((NC, N), jnp.float32),
    mesh=_MESH,
    scratch_types=[
        pltpu.VMEM((NWIN, WIN), jnp.int32),
        pltpu.VMEM((128,), jnp.float32),
        pltpu.VMEM_SHARED((N,), jnp.float32),
        pltpu.SemaphoreType.DMA,
    ],
)


def _scat_body(g_hbm, row_hbm, col_hbm, zeros_hbm, out_hbm,
               rowv, colv, buf0, buf1, acc_sh, sem0, sem1):
    cid = lax.axis_index("c")
    sid = lax.axis_index("s")
    wid = sid * NC + cid

    r0 = sid * ROWS_PER_TILE

    @pl.when(sid < N // ROWS_PER_TILE)
    def _():
        pltpu.sync_copy(zeros_hbm.at[pl.ds(r0, ROWS_PER_TILE)],
                        acc_sh.at[pl.ds(r0, ROWS_PER_TILE)])

    pltpu.sync_copy(row_hbm.at[wid], rowv)
    pltpu.sync_copy(col_hbm.at[wid], colv)
    plsc.subcore_barrier()

    def cidx(w):
        return colv.at[pl.ds(w * WIN, WIN)]

    pltpu.async_copy(g_hbm.at[cidx(0)], buf0, sem0)
    pltpu.async_copy(g_hbm.at[cidx(1)], buf1, sem1)

    def body(w2, carry):
        for k, (buf, sem) in enumerate(((buf0, sem0), (buf1, sem1))):
            w = w2 * 2 + k
            pltpu.make_async_copy(g_hbm.at[cidx(w)], buf, sem).wait()
            pltpu.sync_copy(buf, acc_sh.at[rowv.at[w]], add=True)

            @pl.when(w + 2 < NWIN)
            def _():
                pltpu.async_copy(g_hbm.at[cidx(w + 2)], buf, sem)

        return carry

    lax.fori_loop(0, NWIN // 2, body, 0)
    pltpu.make_async_copy(g_hbm.at[cidx(NWIN - 1)], buf0, sem0).wait()
    pltpu.sync_copy(buf0, acc_sh.at[rowv.at[NWIN - 1]], add=True)
    plsc.subcore_barrier()

    @pl.when(sid < N // ROWS_PER_TILE)
    def _():
        pltpu.sync_copy(acc_sh.at[pl.ds(r0, ROWS_PER_TILE)],
                        out_hbm.at[cid, pl.ds(r0, ROWS_PER_TILE)])


_scat = pl.kernel(
    _scat_body,
    out_type=jax.ShapeDtypeStruct((NC, N, D), jnp.float32),
    mesh=_MESH,
    scratch_types=[
        pltpu.VMEM((NWIN, WIN), jnp.int32),
        pltpu.VMEM((EPW,), jnp.int32),
        pltpu.VMEM((WIN, D), jnp.float32),
        pltpu.VMEM((WIN, D), jnp.float32),
        pltpu.VMEM_SHARED((N, D), jnp.float32),
        pltpu.SemaphoreType.DMA,
        pltpu.SemaphoreType.DMA,
    ],
)


BR = 1000


def _linear_body(cnt_ref, x_ref, wt_ref, b_ref, g_ref):
    deg = cnt_ref[:, 0:1] + cnt_ref[:, 1:2] + 1.0
    dis = lax.rsqrt(deg)
    h = jnp.dot(x_ref[...], wt_ref[...],
                preferred_element_type=jnp.float32) + b_ref[...]
    g_ref[...] = h * dis


_linear = pl.pallas_call(
    _linear_body,
    grid=(N // BR,),
    in_specs=[
        pl.BlockSpec((BR, 2), lambda i: (i, 0)),
        pl.BlockSpec((BR, D), lambda i: (i, 0)),
        pl.BlockSpec((D, D), lambda i: (0, 0)),
        pl.BlockSpec((1, D), lambda i: (0, 0)),
    ],
    out_specs=pl.BlockSpec((BR, D), lambda i: (i, 0)),
    out_shape=jax.ShapeDtypeStruct((N, D), jnp.float32),
)


def _final_body(cnt_ref, p_ref, g_ref, o_ref):
    deg = cnt_ref[:, 0:1] + cnt_ref[:, 1:2] + 1.0
    dis = lax.rsqrt(deg)
    p = p_ref[...]
    o_ref[...] = dis * (p[0] + p[1] + g_ref[...])


_final = pl.pallas_call(
    _final_body,
    grid=(N // BR,),
    in_specs=[
        pl.BlockSpec((BR, 2), lambda i: (i, 0)),
        pl.BlockSpec((NC, BR, D), lambda i: (0, i, 0)),
        pl.BlockSpec((BR, D), lambda i: (i, 0)),
    ],
    out_specs=pl.BlockSpec((BR, D), lambda i: (i, 0)),
    out_shape=jax.ShapeDtypeStruct((N, D), jnp.float32),
)


def kernel(x, edge_index, W, b):
    ei = edge_index.astype(jnp.int32)
    row = ei[0].reshape(NW, NWIN, WIN)
    col = ei[1].reshape(NW, NWIN, WIN)
    col_flat = ei[1].reshape(NW, EPW)
    zeros1 = jnp.zeros((N,), jnp.float32)
    zeros2 = jnp.zeros((N, D), jnp.float32)

    counts = _deg(col, zeros1)
    cnt_t = counts.T
    g = _linear(cnt_t, x, W.T, b.reshape(1, D))
    P = _scat(g, row, col_flat, zeros2)
    return _final(cnt_t, P, g)
